# baseline (speedup 1.0000x reference)
import jax
import jax.numpy as jnp
from jax.experimental import pallas as pl
from jax.experimental.pallas import tpu as pltpu

N = 2500
K = 50
T = 60
F = 5
H = 128
B = 16
G4 = 4 * H
BOUNDS = (0, 800, 1500, 2100, 2500)

_DNT = (((1,), (1,)), ((), ()))


def _dott(a, b):
    return jax.lax.dot_general(a, b, _DNT, preferred_element_type=jnp.float32,
                               precision=jax.lax.Precision.HIGHEST)


def _fused_kernel(x2d_ref, wih0_ref, whh0_ref, b0a_ref, b0b_ref,
                  wih1_ref, whh1_ref, b1a_ref, b1b_ref,
                  wp1_ref, bp1_ref, wp2_ref, bp2_ref,
                  sens_ref, thr_ref, cw_ref,
                  wi1_ref, bi1_ref, wi2_ref, bi2_ref, wi3_ref, bi3_ref,
                  whead_ref, bhead_ref,
                  heads_ref, overall_ref, acts_ref,
                  xp_ref):
    f32 = jnp.float32
    b0 = b0a_ref[:] + b0b_ref[:]
    b1 = b1a_ref[:] + b1b_ref[:]
    xp_ref[:] = jnp.dot(x2d_ref[:], wih0_ref[:], preferred_element_type=f32) + b0

    whh0 = whh0_ref[:]
    wih1 = wih1_ref[:]
    whh1 = whh1_ref[:]

    def step(t, carry):
        h0, c0, h1, c1 = carry
        g0 = xp_ref[pl.ds(t * B, B), :] + jnp.dot(h0, whh0, preferred_element_type=f32)
        i0 = jax.nn.sigmoid(g0[:, :H])
        f0 = jax.nn.sigmoid(g0[:, H:2 * H])
        gg0 = jnp.tanh(g0[:, 2 * H:3 * H])
        o0 = jax.nn.sigmoid(g0[:, 3 * H:])
        c0 = f0 * c0 + i0 * gg0
        h0 = o0 * jnp.tanh(c0)
        g1 = (jnp.dot(h0, wih1, preferred_element_type=f32)
              + jnp.dot(h1, whh1, preferred_element_type=f32) + b1)
        i1 = jax.nn.sigmoid(g1[:, :H])
        f1 = jax.nn.sigmoid(g1[:, H:2 * H])
        gg1 = jnp.tanh(g1[:, 2 * H:3 * H])
        o1 = jax.nn.sigmoid(g1[:, 3 * H:])
        c1 = f1 * c1 + i1 * gg1
        h1 = o1 * jnp.tanh(c1)
        return h0, c0, h1, c1

    z = jnp.zeros((B, H), f32)
    _, _, h1, _ = jax.lax.fori_loop(0, T, step, (z, z, z, z))

    p = jax.nn.relu(jnp.dot(h1, wp1_ref[:], preferred_element_type=f32) + bp1_ref[:])
    base = jnp.tanh(jnp.dot(p, wp2_ref[:], preferred_element_type=f32) + bp2_ref[:])

    S = jnp.sum(base, axis=1, keepdims=True)
    C = _dott(jnp.ones((1, K), f32), cw_ref[:])
    s = S * (C * sens_ref[:])
    sm = s - thr_ref[:]
    nidx = jax.lax.broadcasted_iota(jnp.int32, (1, N), 1)
    neuron = jnp.where(nidx < BOUNDS[1], jax.nn.sigmoid(sm),
             jnp.where(nidx < BOUNDS[2], jnp.tanh(s),
             jnp.where(nidx < BOUNDS[3], jax.nn.relu(sm),
                       jax.nn.sigmoid(s))))

    h = jax.nn.relu(_dott(neuron, wi1_ref[:]) + bi1_ref[:])
    h = jax.nn.relu(_dott(h, wi2_ref[:]) + bi2_ref[:])
    integ = jnp.tanh(_dott(h, wi3_ref[:]) + bi3_ref[:])
    heads = _dott(integ, whead_ref[:]) + bhead_ref[:]
    heads_ref[:] = heads
    overall_ref[:] = jax.nn.sigmoid(heads[:, 14:15])

    cols = []
    for j in range(4):
        lo, hi = BOUNDS[j], BOUNDS[j + 1]
        m = (nidx >= lo) & (nidx < hi)
        cols.append(jnp.sum(jnp.where(m, neuron, 0.0), axis=1, keepdims=True)
                    * (1.0 / (hi - lo)))
    acts_ref[:] = jnp.concatenate(cols, axis=1)


def kernel(x, W_ih0, W_hh0, b_ih0, b_hh0, W_ih1, W_hh1, b_ih1, b_hh1,
           Wp1, bp1, Wp2, bp2, sens, thr, conn_w, conn_idx,
           Wi1, bi1, Wi2, bi2, Wi3, bi3, Wt, bt, Wpat, bpat,
           Wk, bk, Wv, bv, Ws, bs):
    f32 = jnp.float32
    r = lambda v: v.reshape(1, -1)
    x2d = jnp.transpose(x, (1, 0, 2)).reshape(T * B, F)
    whead = jnp.concatenate([Wt, Wpat, Wk, Wv, Ws], axis=0)
    bhead = jnp.concatenate([bt, bpat, bk, bv, bs]).reshape(1, 15)

    heads, overall, acts = pl.pallas_call(
        _fused_kernel,
        out_shape=[
            jax.ShapeDtypeStruct((B, 15), f32),
            jax.ShapeDtypeStruct((B, 1), f32),
            jax.ShapeDtypeStruct((B, 4), f32),
        ],
        scratch_shapes=[pltpu.VMEM((T * B, G4), f32)],
    )(x2d, W_ih0.T, W_hh0.T, r(b_ih0), r(b_hh0),
      W_ih1.T, W_hh1.T, r(b_ih1), r(b_hh1),
      Wp1.T, r(bp1), Wp2.T, r(bp2),
      r(sens), r(thr), conn_w,
      Wi1, r(bi1), Wi2, r(bi2), Wi3, r(bi3),
      whead, bhead)

    trend = heads[:, 0:3]
    pattern = heads[:, 3:9]
    key_levels = heads[:, 9:13]
    vol = heads[:, 13:14]
    conf = heads[:, 14:15]
    overall1 = overall[:, 0]
    return (trend, pattern, key_levels, vol, conf, overall1,
            acts[:, 0], acts[:, 1], acts[:, 2], acts[:, 3])


# speedup vs baseline: 1.1842x; 1.1842x over previous
import jax
import jax.numpy as jnp
from jax.experimental import pallas as pl
from jax.experimental.pallas import tpu as pltpu

N = 2500
NP = 2560
K = 50
T = 60
F = 5
H = 128
B = 16
G4 = 4 * H
BOUNDS = (0, 800, 1500, 2100, 2500)


def _gates(g, c):
    i = jax.nn.sigmoid(g[:, :H])
    f = jax.nn.sigmoid(g[:, H:2 * H])
    gg = jnp.tanh(g[:, 2 * H:3 * H])
    o = jax.nn.sigmoid(g[:, 3 * H:])
    c = f * c + i * gg
    h = o * jnp.tanh(c)
    return h, c


def _fused_kernel(x2d_ref, wih0T_ref, whh0T_ref, b0a_ref, b0b_ref,
                  w1T_ref, b1a_ref, b1b_ref,
                  wp1T_ref, bp1_ref, wp2T_ref, bp2_ref,
                  sens_ref, thr_ref, cwT_ref,
                  wi1T_ref, bi1_ref, wi2T_ref, bi2_ref, wi3T_ref, bi3_ref,
                  wheadT_ref, bhead_ref,
                  heads_ref, overall_ref, acts_ref,
                  xp_ref):
    f32 = jnp.float32
    b0 = b0a_ref[:] + b0b_ref[:]
    b1 = b1a_ref[:] + b1b_ref[:]
    xp_ref[:] = jnp.dot(x2d_ref[:], wih0T_ref[:], preferred_element_type=f32) + b0

    whh0 = whh0T_ref[:]
    w1 = w1T_ref[:]

    def l0_step(t, h0, c0):
        g0 = xp_ref[pl.ds(t * B, B), :] + jnp.dot(h0, whh0, preferred_element_type=f32)
        return _gates(g0, c0)

    def l1_step(h0, h1, c1):
        hcat = jnp.concatenate([h0, h1], axis=1)
        g1 = jnp.dot(hcat, w1, preferred_element_type=f32) + b1
        return _gates(g1, c1)

    z = jnp.zeros((B, H), f32)
    h0, c0 = _gates(xp_ref[0:B, :], z)

    def step(t, carry):
        h0, c0, h1, c1 = carry
        nh0, nc0 = l0_step(t + 1, h0, c0)
        h1, c1 = l1_step(h0, h1, c1)
        return nh0, nc0, h1, c1

    h0, c0, h1, c1 = jax.lax.fori_loop(0, T - 1, step, (h0, c0, z, z))
    h1, c1 = l1_step(h0, h1, c1)

    p = jax.nn.relu(jnp.dot(h1, wp1T_ref[:], preferred_element_type=f32) + bp1_ref[:])
    base = jnp.tanh(jnp.dot(p, wp2T_ref[:], preferred_element_type=f32) + bp2_ref[:])

    S = jnp.sum(base, axis=1, keepdims=True)
    C = jnp.sum(cwT_ref[:], axis=0, keepdims=True)
    s = S * (C * sens_ref[:])
    sm = s - thr_ref[:]
    nidx = jax.lax.broadcasted_iota(jnp.int32, (1, NP), 1)
    neuron = jnp.where(nidx < BOUNDS[1], jax.nn.sigmoid(sm),
             jnp.where(nidx < BOUNDS[2], jnp.tanh(s),
             jnp.where(nidx < BOUNDS[3], jax.nn.relu(sm),
                       jax.nn.sigmoid(s))))

    h = jax.nn.relu(jnp.dot(neuron, wi1T_ref[:], preferred_element_type=f32) + bi1_ref[:])
    h = jax.nn.relu(jnp.dot(h, wi2T_ref[:], preferred_element_type=f32) + bi2_ref[:])
    integ = jnp.tanh(jnp.dot(h, wi3T_ref[:], preferred_element_type=f32) + bi3_ref[:])
    heads = jnp.dot(integ, wheadT_ref[:], preferred_element_type=f32) + bhead_ref[:]
    heads_ref[:] = heads
    overall_ref[:] = jax.nn.sigmoid(heads[:, 14:15])

    cols = []
    for j in range(4):
        lo, hi = BOUNDS[j], BOUNDS[j + 1]
        m = (nidx >= lo) & (nidx < hi)
        cols.append(jnp.sum(jnp.where(m, neuron, 0.0), axis=1, keepdims=True)
                    * (1.0 / (hi - lo)))
    acts_ref[:] = jnp.concatenate(cols, axis=1)


def kernel(x, W_ih0, W_hh0, b_ih0, b_hh0, W_ih1, W_hh1, b_ih1, b_hh1,
           Wp1, bp1, Wp2, bp2, sens, thr, conn_w, conn_idx,
           Wi1, bi1, Wi2, bi2, Wi3, bi3, Wt, bt, Wpat, bpat,
           Wk, bk, Wv, bv, Ws, bs):
    f32 = jnp.float32
    r = lambda v: v.reshape(1, -1)
    x2d = jnp.transpose(x, (1, 0, 2)).reshape(T * B, F)
    w1T = jnp.concatenate([W_ih1.T, W_hh1.T], axis=0)
    cwT = jnp.pad(conn_w.T, ((0, 0), (0, NP - N)))
    sens_p = jnp.pad(sens.reshape(1, N), ((0, 0), (0, NP - N)))
    thr_p = jnp.pad(thr.reshape(1, N), ((0, 0), (0, NP - N)))
    wi1T = jnp.pad(Wi1.T, ((0, NP - N), (0, 0)))
    wheadT = jnp.concatenate([Wt, Wpat, Wk, Wv, Ws], axis=0).T
    bhead = jnp.concatenate([bt, bpat, bk, bv, bs]).reshape(1, 15)

    heads, overall, acts = pl.pallas_call(
        _fused_kernel,
        out_shape=[
            jax.ShapeDtypeStruct((B, 15), f32),
            jax.ShapeDtypeStruct((B, 1), f32),
            jax.ShapeDtypeStruct((B, 4), f32),
        ],
        scratch_shapes=[pltpu.VMEM((T * B, G4), f32)],
    )(x2d, W_ih0.T, W_hh0.T, r(b_ih0), r(b_hh0),
      w1T, r(b_ih1), r(b_hh1),
      Wp1.T, r(bp1), Wp2.T, r(bp2),
      sens_p, thr_p, cwT,
      wi1T, r(bi1), Wi2.T, r(bi2), Wi3.T, r(bi3),
      wheadT, bhead)

    trend = heads[:, 0:3]
    pattern = heads[:, 3:9]
    key_levels = heads[:, 9:13]
    vol = heads[:, 13:14]
    conf = heads[:, 14:15]
    overall1 = overall[:, 0]
    return (trend, pattern, key_levels, vol, conf, overall1,
            acts[:, 0], acts[:, 1], acts[:, 2], acts[:, 3])
